# Initial kernel scaffold; baseline (speedup 1.0000x reference)
import jax
import jax.numpy as jnp
from jax.experimental import pallas as pl


def kernel(x, edge_index, W1, b1, W2, b2):
    raise NotImplementedError("write your pallas kernel here")


# speedup vs baseline: 48.9436x; 48.9436x over previous
import functools

import jax
import jax.numpy as jnp
from jax import lax
from jax.experimental import pallas as pl
from jax.experimental.pallas import tpu as pltpu
from jax.experimental.pallas import tpu_sc as plsc

N = 10000
E = 320000
D_IN = 128
HID = 16

NC = 2
NS = 16
NW = NC * NS

N_PAD = 10240
NPT = N_PAD // NS
EW = 128
ER = -(-E // (EW * NW))
ER = -(-ER // 8) * 8
E_PAD = ER * EW * NW

_mesh = plsc.VectorSubcoreMesh(core_axis_name="c", subcore_axis_name="s")

_sc_params = pltpu.CompilerParams(use_tc_tiling_on_sc=False)


@functools.partial(
    pl.kernel,
    mesh=_mesh,
    compiler_params=_sc_params,
    out_type=jax.ShapeDtypeStruct((NC, N_PAD), jnp.float32),
    scratch_types=[
        pltpu.VMEM((ER, EW), jnp.int32),
        pltpu.VMEM((EW,), jnp.float32),
        pltpu.VMEM_SHARED((N_PAD,), jnp.float32),
    ],
)
def _sc_degree(col_hbm, ones_hbm, zeros1_hbm, out_hbm, col_v, ones_v, deg_sh):
    c = lax.axis_index("c")
    s = lax.axis_index("s")
    wid = c * NS + s
    noff = s * NPT
    pltpu.sync_copy(zeros1_hbm.at[pl.ds(noff, NPT)], deg_sh.at[pl.ds(noff, NPT)])
    pltpu.sync_copy(ones_hbm, ones_v)
    pltpu.sync_copy(col_hbm.at[pl.ds(wid * ER, ER)], col_v)
    plsc.subcore_barrier()

    def body(j, carry):
        pltpu.sync_copy(ones_v, deg_sh.at[col_v.at[j]], add=True)
        return carry

    lax.fori_loop(0, ER, body, 0)
    plsc.subcore_barrier()
    pltpu.sync_copy(deg_sh.at[pl.ds(noff, NPT)], out_hbm.at[c, pl.ds(noff, NPT)])


@functools.partial(
    pl.kernel,
    mesh=_mesh,
    compiler_params=_sc_params,
    out_type=jax.ShapeDtypeStruct((NC, N_PAD, HID), jnp.float32),
    scratch_types=[
        pltpu.VMEM((ER, EW), jnp.int32),
        pltpu.VMEM((ER, EW), jnp.int32),
        pltpu.VMEM((EW, HID), jnp.float32),
        pltpu.VMEM_SHARED((N_PAD, HID), jnp.float32),
        pltpu.VMEM_SHARED((N_PAD, HID), jnp.float32),
        pltpu.SemaphoreType.DMA,
    ],
)
def _sc_agg(s_hbm, row_hbm, col_hbm, zeros_hbm, out_hbm,
            row_v, col_v, gbuf, s_sh, agg_sh, sem):
    c = lax.axis_index("c")
    s = lax.axis_index("s")
    wid = c * NS + s
    noff = s * NPT
    pltpu.sync_copy(s_hbm.at[pl.ds(noff, NPT)], s_sh.at[pl.ds(noff, NPT)])
    pltpu.sync_copy(zeros_hbm.at[pl.ds(noff, NPT)], agg_sh.at[pl.ds(noff, NPT)])
    pltpu.sync_copy(row_hbm.at[pl.ds(wid * ER, ER)], row_v)
    pltpu.sync_copy(col_hbm.at[pl.ds(wid * ER, ER)], col_v)
    plsc.subcore_barrier()

    def body(j, carry):
        pltpu.async_copy(s_sh.at[row_v.at[j]], gbuf, sem).wait()
        pltpu.sync_copy(gbuf, agg_sh.at[col_v.at[j]], add=True)
        return carry

    lax.fori_loop(0, ER, body, 0)
    plsc.subcore_barrier()
    pltpu.sync_copy(agg_sh.at[pl.ds(noff, NPT)], out_hbm.at[c, pl.ds(noff, NPT)])


def _tc1_body(deg_ref, x_ref, w1_ref, s1_ref, dinv_ref):
    deg = deg_ref[0] + deg_ref[1] + 1.0
    dinv = lax.rsqrt(deg)
    h = jnp.dot(x_ref[...], w1_ref[...], preferred_element_type=jnp.float32)
    s1_ref[...] = h * dinv
    dinv_ref[...] = dinv


def _tc2_body(agg_ref, s1_ref, dinv_ref, b1_ref, w2_ref, s2_ref):
    a = agg_ref[0] + agg_ref[1] + s1_ref[...]
    h = jnp.maximum(a * dinv_ref[...] + b1_ref[...], 0.0)
    s2_ref[...] = jnp.dot(h, w2_ref[...],
                          preferred_element_type=jnp.float32) * dinv_ref[...]


def _tc3_body(agg_ref, s2_ref, dinv_ref, b2_ref, out_ref):
    o = (agg_ref[0] + agg_ref[1] + s2_ref[...]) * dinv_ref[...] + b2_ref[...]
    m = jnp.max(o, axis=1, keepdims=True)
    e = jnp.exp(o - m)
    lse = jnp.log(jnp.sum(e, axis=1, keepdims=True))
    out_ref[...] = o - m - lse


def kernel(x, edge_index, W1, b1, W2, b2):
    row = edge_index[0]
    col = edge_index[1]
    pad = E_PAD - E
    row_p = jnp.concatenate([row, jnp.zeros((pad,), jnp.int32)]).reshape(
        NW * ER, EW)
    col_p = jnp.concatenate([col, jnp.full((pad,), N, jnp.int32)]).reshape(
        NW * ER, EW)

    x_p = jnp.zeros((N_PAD, D_IN), jnp.float32).at[:N].set(x)
    zeros16 = jnp.zeros((N_PAD, HID), jnp.float32)
    zeros1 = jnp.zeros((N_PAD,), jnp.float32)
    ones = jnp.ones((EW,), jnp.float32)

    deg2 = _sc_degree(col_p, ones, zeros1)
    deg3 = deg2.reshape(NC, N_PAD, 1)

    s1, dinv = pl.pallas_call(
        _tc1_body,
        out_shape=[
            jax.ShapeDtypeStruct((N_PAD, HID), jnp.float32),
            jax.ShapeDtypeStruct((N_PAD, 1), jnp.float32),
        ],
    )(deg3, x_p, W1)

    agg1 = _sc_agg(s1, row_p, col_p, zeros16)

    s2 = pl.pallas_call(
        _tc2_body,
        out_shape=jax.ShapeDtypeStruct((N_PAD, HID), jnp.float32),
    )(agg1, s1, dinv, b1.reshape(1, HID), W2)

    agg2 = _sc_agg(s2, row_p, col_p, zeros16)

    out = pl.pallas_call(
        _tc3_body,
        out_shape=jax.ShapeDtypeStruct((N_PAD, HID), jnp.float32),
    )(agg2, s2, dinv, b2.reshape(1, HID))

    return out[:N]


# speedup vs baseline: 57.9023x; 1.1830x over previous
import functools

import jax
import jax.numpy as jnp
from jax import lax
from jax.experimental import pallas as pl
from jax.experimental.pallas import tpu as pltpu
from jax.experimental.pallas import tpu_sc as plsc

N = 10000
E = 320000
D_IN = 128
HID = 16

NC = 2
NS = 16
NW = NC * NS

N_PAD = 10240
NPT = N_PAD // NS
NCH = 4
EPT = -(-E // NW)
EPT = -(-EPT // (8 * NCH)) * (8 * NCH)
E_PAD = EPT * NW
CH = EPT // NCH

_mesh = plsc.VectorSubcoreMesh(core_axis_name="c", subcore_axis_name="s")

_sc_params = pltpu.CompilerParams(use_tc_tiling_on_sc=False)


@functools.partial(
    pl.kernel,
    mesh=_mesh,
    compiler_params=_sc_params,
    out_type=jax.ShapeDtypeStruct((NC, N_PAD), jnp.float32),
    scratch_types=[
        pltpu.VMEM((EPT,), jnp.int32),
        pltpu.VMEM((EPT,), jnp.float32),
        pltpu.VMEM_SHARED((N_PAD,), jnp.float32),
    ],
)
def _sc_degree(col_hbm, ones_hbm, zeros1_hbm, out_hbm, col_v, ones_v, deg_sh):
    c = lax.axis_index("c")
    s = lax.axis_index("s")
    wid = c * NS + s
    noff = s * NPT
    pltpu.sync_copy(zeros1_hbm.at[pl.ds(noff, NPT)], deg_sh.at[pl.ds(noff, NPT)])
    pltpu.sync_copy(ones_hbm, ones_v)
    pltpu.sync_copy(col_hbm.at[pl.ds(wid * EPT, EPT)], col_v)
    plsc.subcore_barrier()
    pltpu.sync_copy(ones_v, deg_sh.at[col_v], add=True)
    plsc.subcore_barrier()
    pltpu.sync_copy(deg_sh.at[pl.ds(noff, NPT)], out_hbm.at[c, pl.ds(noff, NPT)])


@functools.partial(
    pl.kernel,
    mesh=_mesh,
    compiler_params=_sc_params,
    out_type=jax.ShapeDtypeStruct((NC, N_PAD, HID), jnp.float32),
    scratch_types=[
        pltpu.VMEM((CH,), jnp.int32),
        pltpu.VMEM((CH,), jnp.int32),
        pltpu.VMEM((CH,), jnp.int32),
        pltpu.VMEM((CH,), jnp.int32),
        pltpu.VMEM((CH, HID), jnp.float32),
        pltpu.VMEM((CH, HID), jnp.float32),
        pltpu.VMEM_SHARED((N_PAD, HID), jnp.float32),
        pltpu.VMEM_SHARED((N_PAD, HID), jnp.float32),
        pltpu.SemaphoreType.DMA,
        pltpu.SemaphoreType.DMA,
    ],
)
def _sc_agg(s_hbm, row_hbm, col_hbm, zeros_hbm, out_hbm,
            row_v0, col_v0, row_v1, col_v1, gb0, gb1, s_sh, agg_sh,
            sem0, sem1):
    c = lax.axis_index("c")
    s = lax.axis_index("s")
    wid = c * NS + s
    noff = s * NPT
    ebase = wid * EPT
    pltpu.sync_copy(s_hbm.at[pl.ds(noff, NPT)], s_sh.at[pl.ds(noff, NPT)])
    pltpu.sync_copy(zeros_hbm.at[pl.ds(noff, NPT)], agg_sh.at[pl.ds(noff, NPT)])
    plsc.subcore_barrier()

    bufs = [(row_v0, col_v0, gb0, sem0), (row_v1, col_v1, gb1, sem1)]

    def load_and_gather(k):
        row_v, col_v, gb, sem = bufs[k % 2]
        off = ebase + k * CH
        pltpu.sync_copy(row_hbm.at[pl.ds(off, CH)], row_v)
        pltpu.sync_copy(col_hbm.at[pl.ds(off, CH)], col_v)
        return pltpu.async_copy(s_sh.at[row_v], gb, sem)

    g = load_and_gather(0)
    for k in range(NCH):
        if k + 1 < NCH:
            g_next = load_and_gather(k + 1)
        g.wait()
        _, col_v, gb, _ = bufs[k % 2]
        pltpu.sync_copy(gb, agg_sh.at[col_v], add=True)
        if k + 1 < NCH:
            g = g_next

    plsc.subcore_barrier()
    pltpu.sync_copy(agg_sh.at[pl.ds(noff, NPT)], out_hbm.at[c, pl.ds(noff, NPT)])


def _tc1_body(deg_ref, x_ref, w1_ref, s1_ref, dinv_ref):
    deg = deg_ref[0] + deg_ref[1] + 1.0
    dinv = lax.rsqrt(deg)
    h = jnp.dot(x_ref[...], w1_ref[...], preferred_element_type=jnp.float32)
    s1_ref[...] = h * dinv
    dinv_ref[...] = dinv


def _tc2_body(agg_ref, s1_ref, dinv_ref, b1_ref, w2_ref, s2_ref):
    a = agg_ref[0] + agg_ref[1] + s1_ref[...]
    h = jnp.maximum(a * dinv_ref[...] + b1_ref[...], 0.0)
    s2_ref[...] = jnp.dot(h, w2_ref[...],
                          preferred_element_type=jnp.float32) * dinv_ref[...]


def _tc3_body(agg_ref, s2_ref, dinv_ref, b2_ref, out_ref):
    o = (agg_ref[0] + agg_ref[1] + s2_ref[...]) * dinv_ref[...] + b2_ref[...]
    m = jnp.max(o, axis=1, keepdims=True)
    e = jnp.exp(o - m)
    lse = jnp.log(jnp.sum(e, axis=1, keepdims=True))
    out_ref[...] = o - m - lse


def kernel(x, edge_index, W1, b1, W2, b2):
    row = edge_index[0]
    col = edge_index[1]
    pad = E_PAD - E
    row_p = jnp.concatenate([row, jnp.zeros((pad,), jnp.int32)])
    col_p = jnp.concatenate([col, jnp.full((pad,), N, jnp.int32)])

    x_p = jnp.zeros((N_PAD, D_IN), jnp.float32).at[:N].set(x)
    zeros16 = jnp.zeros((N_PAD, HID), jnp.float32)
    zeros1 = jnp.zeros((N_PAD,), jnp.float32)
    ones = jnp.ones((EPT,), jnp.float32)

    deg2 = _sc_degree(col_p, ones, zeros1)
    deg3 = deg2.reshape(NC, N_PAD, 1)

    s1, dinv = pl.pallas_call(
        _tc1_body,
        out_shape=[
            jax.ShapeDtypeStruct((N_PAD, HID), jnp.float32),
            jax.ShapeDtypeStruct((N_PAD, 1), jnp.float32),
        ],
    )(deg3, x_p, W1)

    agg1 = _sc_agg(s1, row_p, col_p, zeros16)

    s2 = pl.pallas_call(
        _tc2_body,
        out_shape=jax.ShapeDtypeStruct((N_PAD, HID), jnp.float32),
    )(agg1, s1, dinv, b1.reshape(1, HID), W2)

    agg2 = _sc_agg(s2, row_p, col_p, zeros16)

    out = pl.pallas_call(
        _tc3_body,
        out_shape=jax.ShapeDtypeStruct((N_PAD, HID), jnp.float32),
    )(agg2, s2, dinv, b2.reshape(1, HID))

    return out[:N]


# speedup vs baseline: 93.7193x; 1.6186x over previous
import functools

import jax
import jax.numpy as jnp
from jax import lax
from jax.experimental import pallas as pl
from jax.experimental.pallas import tpu as pltpu
from jax.experimental.pallas import tpu_sc as plsc

N = 10000
E = 320000
D_IN = 128
HID = 16

NC = 2
NS = 16
NW = NC * NS

N_PAD = 10240
NPT = N_PAD // NS
PR = N_PAD // 8
PRV = N // 8
NCH = 5
EPT = E // NW
CH = EPT // NCH

_mesh = plsc.VectorSubcoreMesh(core_axis_name="c", subcore_axis_name="s")

_sc_params = pltpu.CompilerParams(use_tc_tiling_on_sc=False)

def _zrow():
    return jnp.zeros((16,), jnp.float32)


def _orow():
    return jnp.ones((16,), jnp.float32)


@functools.partial(
    pl.kernel,
    mesh=_mesh,
    compiler_params=_sc_params,
    out_type=jax.ShapeDtypeStruct((NC, N_PAD * HID), jnp.float32),
    scratch_types=[
        pltpu.VMEM((EPT,), jnp.int32),
        pltpu.VMEM((EPT,), jnp.float32),
        pltpu.VMEM((NPT,), jnp.float32),
        pltpu.VMEM((NPT * HID,), jnp.int32),
        pltpu.VMEM((NPT * HID,), jnp.float32),
        pltpu.VMEM_SHARED((N_PAD,), jnp.float32),
        pltpu.SemaphoreType.DMA,
    ],
)
def _sc_degree(ei_hbm, out_hbm, col_v, ones_v, zbuf, idxb, degb, deg_sh, sem):
    c = lax.axis_index("c")
    s = lax.axis_index("s")
    wid = c * NS + s
    noff = s * NPT
    gcp = pltpu.async_copy(ei_hbm.at[1, pl.ds(wid * EPT, EPT)], col_v, sem)

    def fill_z(j, carry):
        zbuf[pl.ds(j * 16, 16)] = _zrow()
        return carry

    lax.fori_loop(0, NPT // 16, fill_z, 0)

    def fill_o(j, carry):
        ones_v[pl.ds(j * 16, 16)] = _orow()
        return carry

    lax.fori_loop(0, EPT // 16, fill_o, 0)

    def fill_idx(j, carry):
        idxb[pl.ds(j * 16, 16)] = jnp.full((16,), noff + j, jnp.int32)
        return carry

    lax.fori_loop(0, NPT, fill_idx, 0)

    pltpu.sync_copy(zbuf, deg_sh.at[pl.ds(noff, NPT)])
    gcp.wait()
    plsc.subcore_barrier()
    pltpu.sync_copy(ones_v, deg_sh.at[col_v], add=True)
    plsc.subcore_barrier()
    pltpu.async_copy(deg_sh.at[idxb], degb, sem).wait()
    pltpu.sync_copy(degb, out_hbm.at[c, pl.ds(noff * HID, NPT * HID)])


@functools.partial(
    pl.kernel,
    mesh=_mesh,
    compiler_params=_sc_params,
    out_type=jax.ShapeDtypeStruct((NC, N_PAD, HID), jnp.float32),
    scratch_types=[
        pltpu.VMEM((CH,), jnp.int32),
        pltpu.VMEM((CH,), jnp.int32),
        pltpu.VMEM((CH,), jnp.int32),
        pltpu.VMEM((CH,), jnp.int32),
        pltpu.VMEM((CH, HID), jnp.float32),
        pltpu.VMEM((CH, HID), jnp.float32),
        pltpu.VMEM((NPT, HID), jnp.float32),
        pltpu.VMEM_SHARED((N_PAD, HID), jnp.float32),
        pltpu.VMEM_SHARED((N_PAD, HID), jnp.float32),
        pltpu.SemaphoreType.DMA,
        pltpu.SemaphoreType.DMA,
    ],
)
def _sc_agg(s_hbm, ei_hbm, out_hbm,
            row_v0, col_v0, row_v1, col_v1, gb0, gb1, zbuf, s_sh, agg_sh,
            sem0, sem1):
    c = lax.axis_index("c")
    s = lax.axis_index("s")
    wid = c * NS + s
    noff = s * NPT
    ebase = wid * EPT
    scp = pltpu.async_copy(s_hbm.at[pl.ds(noff, NPT)],
                           s_sh.at[pl.ds(noff, NPT)], sem0)

    def fill_z(j, carry):
        zbuf[j, :] = _zrow()
        return carry

    lax.fori_loop(0, NPT, fill_z, 0)
    scp.wait()
    pltpu.sync_copy(zbuf, agg_sh.at[pl.ds(noff, NPT)])
    plsc.subcore_barrier()

    bufs = [(row_v0, col_v0, gb0, sem0), (row_v1, col_v1, gb1, sem1)]

    def load_and_gather(k):
        row_v, col_v, gb, sem = bufs[k % 2]
        off = ebase + k * CH
        pltpu.sync_copy(ei_hbm.at[0, pl.ds(off, CH)], row_v)
        pltpu.sync_copy(ei_hbm.at[1, pl.ds(off, CH)], col_v)
        return pltpu.async_copy(s_sh.at[row_v], gb, sem)

    g = load_and_gather(0)
    for k in range(NCH):
        if k + 1 < NCH:
            g_next = load_and_gather(k + 1)
        g.wait()
        _, col_v, gb, _ = bufs[k % 2]
        pltpu.sync_copy(gb, agg_sh.at[col_v], add=True)
        if k + 1 < NCH:
            g = g_next

    plsc.subcore_barrier()
    pltpu.sync_copy(agg_sh.at[pl.ds(noff, NPT)], out_hbm.at[c, pl.ds(noff, NPT)])


def _tc1_body(degb_ref, x_ref, w1_ref, s1_ref, dinv_ref):
    dinv = lax.rsqrt(degb_ref[0] + degb_ref[1] + 1.0)
    hp = jnp.dot(x_ref[...], w1_ref[...], preferred_element_type=jnp.float32)
    s1_ref[:PRV, :] = hp * dinv[:PRV, :]
    s1_ref[PRV:, :] = jnp.zeros((PR - PRV, 128), jnp.float32)
    dinv_ref[...] = dinv


def _tc2_body(agg_ref, s1_ref, dinv_ref, b1_ref, w2_ref, s2_ref):
    a = agg_ref[0] + agg_ref[1] + s1_ref[...]
    h = jnp.maximum(a * dinv_ref[...] + b1_ref[...], 0.0)
    s2_ref[...] = jnp.dot(h, w2_ref[...],
                          preferred_element_type=jnp.float32) * dinv_ref[...]


def _tc3_body(agg_ref, s2_ref, dinv_ref, b2_ref, g_ref, out_ref):
    o = (agg_ref[0] + agg_ref[1] + s2_ref[...]) * dinv_ref[...] + b2_ref[...]
    g = g_ref[...]
    mean = jnp.dot(o, g, preferred_element_type=jnp.float32) * (1.0 / HID)
    oc = o - mean
    ssum = jnp.dot(jnp.exp(oc), g, preferred_element_type=jnp.float32)
    out_ref[...] = oc - jnp.log(ssum)


def kernel(x, edge_index, W1, b1, W2, b2):
    degb = _sc_degree(edge_index)
    degb_p = degb.reshape(NC, PR, 128)

    b1t = jnp.tile(b1, 8).reshape(1, 128)
    b2t = jnp.tile(b2, 8).reshape(1, 128)
    w1big = jax.scipy.linalg.block_diag(*([W1] * 8))
    w2big = jax.scipy.linalg.block_diag(*([W2] * 8))
    lane = jnp.arange(128, dtype=jnp.int32)
    gmat = (lane[:, None] // HID == lane[None, :] // HID).astype(jnp.float32)
    x_r = x.reshape(PRV, 8 * D_IN)

    s1p, dinvb = pl.pallas_call(
        _tc1_body,
        out_shape=[
            jax.ShapeDtypeStruct((PR, 128), jnp.float32),
            jax.ShapeDtypeStruct((PR, 128), jnp.float32),
        ],
    )(degb_p, x_r, w1big)

    agg1 = _sc_agg(s1p.reshape(N_PAD, HID), edge_index)

    s2p = pl.pallas_call(
        _tc2_body,
        out_shape=jax.ShapeDtypeStruct((PR, 128), jnp.float32),
    )(agg1.reshape(NC, PR, 128), s1p, dinvb, b1t, w2big)

    agg2 = _sc_agg(s2p.reshape(N_PAD, HID), edge_index)

    out_p = pl.pallas_call(
        _tc3_body,
        out_shape=jax.ShapeDtypeStruct((PR, 128), jnp.float32),
    )(agg2.reshape(NC, PR, 128), s2p, dinvb, b2t, gmat)

    return out_p.reshape(N_PAD, HID)[:N]


# speedup vs baseline: 101.4083x; 1.0820x over previous
import functools

import jax
import jax.numpy as jnp
from jax import lax
from jax.experimental import pallas as pl
from jax.experimental.pallas import tpu as pltpu
from jax.experimental.pallas import tpu_sc as plsc

N = 10000
E = 320000
D_IN = 128
HID = 16

NC = 2
NS = 16
NW = NC * NS

N_PAD = 10240
NPT = N_PAD // NS
PR = N_PAD // 8
PRV = N // 8
NCH = 5
EPT = E // NW
CH = EPT // NCH

_mesh = plsc.VectorSubcoreMesh(core_axis_name="c", subcore_axis_name="s")

_sc_params = pltpu.CompilerParams(use_tc_tiling_on_sc=False)

def _zrow():
    return jnp.zeros((16,), jnp.float32)


def _orow():
    return jnp.ones((16,), jnp.float32)


@functools.partial(
    pl.kernel,
    mesh=_mesh,
    compiler_params=_sc_params,
    out_type=jax.ShapeDtypeStruct((NC, N_PAD * HID), jnp.float32),
    scratch_types=[
        pltpu.VMEM((EPT,), jnp.int32),
        pltpu.VMEM((EPT,), jnp.float32),
        pltpu.VMEM((NPT,), jnp.float32),
        pltpu.VMEM((NPT * HID,), jnp.int32),
        pltpu.VMEM((NPT * HID,), jnp.float32),
        pltpu.VMEM_SHARED((N_PAD,), jnp.float32),
        pltpu.SemaphoreType.DMA,
    ],
)
def _sc_degree(ei_hbm, out_hbm, col_v, ones_v, zbuf, idxb, degb, deg_sh, sem):
    c = lax.axis_index("c")
    s = lax.axis_index("s")
    wid = c * NS + s
    noff = s * NPT
    gcp = pltpu.async_copy(ei_hbm.at[1, pl.ds(wid * EPT, EPT)], col_v, sem)

    def fill_z(j, carry):
        for u in range(8):
            zbuf[pl.ds(j * 128 + u * 16, 16)] = _zrow()
        return carry

    lax.fori_loop(0, NPT // 128, fill_z, 0)

    def fill_o(j, carry):
        for u in range(8):
            ones_v[pl.ds(j * 128 + u * 16, 16)] = _orow()
        return carry

    lax.fori_loop(0, EPT // 128, fill_o, 0)
    for r in range(EPT - EPT % 128, EPT, 16):
        ones_v[pl.ds(r, 16)] = _orow()

    def fill_idx(j, carry):
        for u in range(8):
            idxb[pl.ds((j * 8 + u) * 16, 16)] = jnp.full((16,), noff + j * 8 + u,
                                                         jnp.int32)
        return carry

    lax.fori_loop(0, NPT // 8, fill_idx, 0)

    pltpu.sync_copy(zbuf, deg_sh.at[pl.ds(noff, NPT)])
    gcp.wait()
    plsc.subcore_barrier()
    pltpu.sync_copy(ones_v, deg_sh.at[col_v], add=True)
    plsc.subcore_barrier()
    pltpu.async_copy(deg_sh.at[idxb], degb, sem).wait()
    pltpu.sync_copy(degb, out_hbm.at[c, pl.ds(noff * HID, NPT * HID)])


@functools.partial(
    pl.kernel,
    mesh=_mesh,
    compiler_params=_sc_params,
    out_type=jax.ShapeDtypeStruct((NC, N_PAD, HID), jnp.float32),
    scratch_types=[
        pltpu.VMEM((CH,), jnp.int32),
        pltpu.VMEM((CH,), jnp.int32),
        pltpu.VMEM((CH,), jnp.int32),
        pltpu.VMEM((CH,), jnp.int32),
        pltpu.VMEM((CH, HID), jnp.float32),
        pltpu.VMEM((CH, HID), jnp.float32),
        pltpu.VMEM((NPT, HID), jnp.float32),
        pltpu.VMEM_SHARED((N_PAD, HID), jnp.float32),
        pltpu.SemaphoreType.DMA,
        pltpu.SemaphoreType.DMA,
    ],
)
def _sc_agg(s_hbm, ei_hbm, out_hbm,
            row_v0, col_v0, row_v1, col_v1, gb0, gb1, zbuf, agg_sh,
            sem0, sem1):
    c = lax.axis_index("c")
    s = lax.axis_index("s")
    wid = c * NS + s
    noff = s * NPT
    ebase = wid * EPT

    def fill_z(j, carry):
        for u in range(8):
            zbuf[j * 8 + u, :] = _zrow()
        return carry

    lax.fori_loop(0, NPT // 8, fill_z, 0)
    pltpu.sync_copy(zbuf, agg_sh.at[pl.ds(noff, NPT)])
    plsc.subcore_barrier()

    bufs = [(row_v0, col_v0, gb0, sem0), (row_v1, col_v1, gb1, sem1)]

    def load_and_gather(k):
        row_v, col_v, gb, sem = bufs[k % 2]
        off = ebase + k * CH
        pltpu.sync_copy(ei_hbm.at[0, pl.ds(off, CH)], row_v)
        pltpu.sync_copy(ei_hbm.at[1, pl.ds(off, CH)], col_v)
        return pltpu.async_copy(s_hbm.at[row_v], gb, sem)

    g = load_and_gather(0)
    for k in range(NCH):
        if k + 1 < NCH:
            g_next = load_and_gather(k + 1)
        g.wait()
        _, col_v, gb, _ = bufs[k % 2]
        pltpu.sync_copy(gb, agg_sh.at[col_v], add=True)
        if k + 1 < NCH:
            g = g_next

    plsc.subcore_barrier()
    pltpu.sync_copy(agg_sh.at[pl.ds(noff, NPT)], out_hbm.at[c, pl.ds(noff, NPT)])


def _tc1_body(degb_ref, x_ref, w1_ref, s1_ref, dinv_ref):
    dinv = lax.rsqrt(degb_ref[0] + degb_ref[1] + 1.0)
    hp = jnp.dot(x_ref[...], w1_ref[...], preferred_element_type=jnp.float32)
    s1_ref[:PRV, :] = hp * dinv[:PRV, :]
    s1_ref[PRV:, :] = jnp.zeros((PR - PRV, 128), jnp.float32)
    dinv_ref[...] = dinv


def _tc2_body(agg_ref, s1_ref, dinv_ref, b1_ref, w2_ref, s2_ref):
    a = agg_ref[0] + agg_ref[1] + s1_ref[...]
    h = jnp.maximum(a * dinv_ref[...] + b1_ref[...], 0.0)
    s2_ref[...] = jnp.dot(h, w2_ref[...],
                          preferred_element_type=jnp.float32) * dinv_ref[...]


def _tc3_body(agg_ref, s2_ref, dinv_ref, b2_ref, g_ref, out_ref):
    o = (agg_ref[0] + agg_ref[1] + s2_ref[...]) * dinv_ref[...] + b2_ref[...]
    g = g_ref[...]
    mean = jnp.dot(o, g, preferred_element_type=jnp.float32) * (1.0 / HID)
    oc = o - mean
    ssum = jnp.dot(jnp.exp(oc), g, preferred_element_type=jnp.float32)
    out_ref[...] = oc - jnp.log(ssum)


def kernel(x, edge_index, W1, b1, W2, b2):
    degb = _sc_degree(edge_index)
    degb_p = degb.reshape(NC, PR, 128)

    b1t = jnp.tile(b1, 8).reshape(1, 128)
    b2t = jnp.tile(b2, 8).reshape(1, 128)
    w1big = jax.scipy.linalg.block_diag(*([W1] * 8))
    w2big = jax.scipy.linalg.block_diag(*([W2] * 8))
    lane = jnp.arange(128, dtype=jnp.int32)
    gmat = (lane[:, None] // HID == lane[None, :] // HID).astype(jnp.float32)
    x_r = x.reshape(PRV, 8 * D_IN)

    s1p, dinvb = pl.pallas_call(
        _tc1_body,
        out_shape=[
            jax.ShapeDtypeStruct((PR, 128), jnp.float32),
            jax.ShapeDtypeStruct((PR, 128), jnp.float32),
        ],
    )(degb_p, x_r, w1big)

    agg1 = _sc_agg(s1p.reshape(N_PAD, HID), edge_index)

    s2p = pl.pallas_call(
        _tc2_body,
        out_shape=jax.ShapeDtypeStruct((PR, 128), jnp.float32),
    )(agg1.reshape(NC, PR, 128), s1p, dinvb, b1t, w2big)

    agg2 = _sc_agg(s2p.reshape(N_PAD, HID), edge_index)

    out_p = pl.pallas_call(
        _tc3_body,
        out_shape=jax.ShapeDtypeStruct((PR, 128), jnp.float32),
    )(agg2.reshape(NC, PR, 128), s2p, dinvb, b2t, gmat)

    return out_p.reshape(N_PAD, HID)[:N]
